# baseline (speedup 1.0000x reference)
import functools

import jax
import jax.numpy as jnp
from jax.experimental import pallas as pl

BATCH = 4096
TILE = 256
FEAT = 1024
EMB = 64
NCLS = 1000
NCLS_PAD = 1024
N_TILES = BATCH // TILE


def _mlp_kernel(x_ref, y_ref, embed_ref, w0_ref, b0_ref, w1_ref,
                b1_ref, w2_ref, b2_ref, w3_ref, b3_ref, out_ref):
    yv = y_ref[0, 0, :]
    oh = (yv[:, None] == jax.lax.broadcasted_iota(
        jnp.int32, (TILE, NCLS_PAD), 1)).astype(jnp.float32)
    y_e = jnp.dot(oh, embed_ref[...], preferred_element_type=jnp.float32)
    h = jnp.dot(x_ref[...], w0_ref[0:FEAT, :],
                preferred_element_type=jnp.float32)
    h = h + jnp.dot(y_e, w0_ref[FEAT:FEAT + EMB, :],
                    preferred_element_type=jnp.float32)
    h = h + b0_ref[...]
    h = jnp.maximum(h, 0.2 * h)
    h = jnp.dot(h, w1_ref[...], preferred_element_type=jnp.float32) + b1_ref[...]
    h = jnp.maximum(h, 0.2 * h)
    h = jnp.dot(h, w2_ref[...], preferred_element_type=jnp.float32) + b2_ref[...]
    h = jnp.maximum(h, 0.2 * h)
    o = jnp.dot(h, w3_ref[...], preferred_element_type=jnp.float32)
    out_ref[...] = o + b3_ref[...]


@functools.partial(jax.jit, static_argnames=("interpret",))
def kernel(x, y, embed, W0, b0, W1, b1, W2, b2, W3, b3, interpret=False):
    y3 = y.astype(jnp.int32).reshape(N_TILES, 1, TILE)
    embed_p = jnp.zeros((NCLS_PAD, EMB), jnp.float32).at[:NCLS].set(embed)
    out = pl.pallas_call(
        _mlp_kernel,
        grid=(N_TILES,),
        in_specs=[
            pl.BlockSpec((TILE, FEAT), lambda i: (i, 0)),
            pl.BlockSpec((1, 1, TILE), lambda i: (i, 0, 0)),
            pl.BlockSpec((NCLS_PAD, EMB), lambda i: (0, 0)),
            pl.BlockSpec((FEAT + EMB, 1024), lambda i: (0, 0)),
            pl.BlockSpec((1, 1024), lambda i: (0, 0)),
            pl.BlockSpec((1024, 512), lambda i: (0, 0)),
            pl.BlockSpec((1, 512), lambda i: (0, 0)),
            pl.BlockSpec((512, 256), lambda i: (0, 0)),
            pl.BlockSpec((1, 256), lambda i: (0, 0)),
            pl.BlockSpec((256, 1), lambda i: (0, 0)),
            pl.BlockSpec((1, 1), lambda i: (0, 0)),
        ],
        out_specs=pl.BlockSpec((TILE, 1), lambda i: (i, 0)),
        out_shape=jax.ShapeDtypeStruct((BATCH, 1), jnp.float32),
        interpret=interpret,
    )(x, y3, embed_p, W0, b0.reshape(1, -1), W1, b1.reshape(1, -1),
      W2, b2.reshape(1, -1), W3, b3.reshape(1, 1))
    return out.reshape(BATCH)


# speedup vs baseline: 1.1471x; 1.1471x over previous
import functools

import jax
import jax.numpy as jnp
from jax.experimental import pallas as pl

BATCH = 4096
TILE = 1024
FEAT = 1024
EMB = 64
NCLS = 1000
NCLS_PAD = 1024
N_TILES = BATCH // TILE


def _mlp_kernel(x_ref, y_ref, embed_ref, w0_ref, b0_ref, w1_ref,
                b1_ref, w2_ref, b2_ref, w3_ref, b3_ref, out_ref):
    yv = y_ref[0, 0, :]
    oh = (yv[:, None] == jax.lax.broadcasted_iota(
        jnp.int32, (TILE, NCLS_PAD), 1)).astype(jnp.float32)
    y_e = jnp.dot(oh, embed_ref[...], preferred_element_type=jnp.float32)
    h = jnp.dot(x_ref[...], w0_ref[0:FEAT, :],
                preferred_element_type=jnp.float32)
    h = h + jnp.dot(y_e, w0_ref[FEAT:FEAT + EMB, :],
                    preferred_element_type=jnp.float32)
    h = h + b0_ref[...]
    h = jnp.maximum(h, 0.2 * h)
    h = jnp.dot(h, w1_ref[...], preferred_element_type=jnp.float32) + b1_ref[...]
    h = jnp.maximum(h, 0.2 * h)
    h = jnp.dot(h, w2_ref[...], preferred_element_type=jnp.float32) + b2_ref[...]
    h = jnp.maximum(h, 0.2 * h)
    o = jnp.dot(h, w3_ref[...], preferred_element_type=jnp.float32)
    out_ref[...] = o + b3_ref[...]


@functools.partial(jax.jit, static_argnames=("interpret",))
def kernel(x, y, embed, W0, b0, W1, b1, W2, b2, W3, b3, interpret=False):
    y3 = y.astype(jnp.int32).reshape(N_TILES, 1, TILE)
    embed_p = jnp.zeros((NCLS_PAD, EMB), jnp.float32).at[:NCLS].set(embed)
    out = pl.pallas_call(
        _mlp_kernel,
        grid=(N_TILES,),
        in_specs=[
            pl.BlockSpec((TILE, FEAT), lambda i: (i, 0)),
            pl.BlockSpec((1, 1, TILE), lambda i: (i, 0, 0)),
            pl.BlockSpec((NCLS_PAD, EMB), lambda i: (0, 0)),
            pl.BlockSpec((FEAT + EMB, 1024), lambda i: (0, 0)),
            pl.BlockSpec((1, 1024), lambda i: (0, 0)),
            pl.BlockSpec((1024, 512), lambda i: (0, 0)),
            pl.BlockSpec((1, 512), lambda i: (0, 0)),
            pl.BlockSpec((512, 256), lambda i: (0, 0)),
            pl.BlockSpec((1, 256), lambda i: (0, 0)),
            pl.BlockSpec((256, 1), lambda i: (0, 0)),
            pl.BlockSpec((1, 1), lambda i: (0, 0)),
        ],
        out_specs=pl.BlockSpec((TILE, 1), lambda i: (i, 0)),
        out_shape=jax.ShapeDtypeStruct((BATCH, 1), jnp.float32),
        interpret=interpret,
    )(x, y3, embed_p, W0, b0.reshape(1, -1), W1, b1.reshape(1, -1),
      W2, b2.reshape(1, -1), W3, b3.reshape(1, 1))
    return out.reshape(BATCH)
